# Initial kernel scaffold; baseline (speedup 1.0000x reference)
import jax
import jax.numpy as jnp
from jax.experimental import pallas as pl


def kernel(x, edge_index, edge_type, W0, S0, b0, W1, S1, b1, W2, S2, b2, W3, S3, b3):
    raise NotImplementedError("write your pallas kernel here")


# speedup vs baseline: 2.0192x; 2.0192x over previous
import functools

import jax
import jax.numpy as jnp
from jax import lax
from jax.experimental import pallas as pl
from jax.experimental.pallas import tpu as pltpu
from jax.experimental.pallas import tpu_sc as plsc

N = 10000
E = 160000
R = 8
D = 256
H = 128
RN = R * N

NC = 2
NS = 16
CH = 128

P_SUB = -(-E // (NS * CH)) * CH
E_PAD = NS * P_SUB
N_CH = P_SUB // CH

ACC = 10496
ROWS_PER_TILE = ACC // NS
BN = 1000


def _hrk_body(h_ref, w_ref, o_ref):
    res = jnp.dot(h_ref[...], w_ref[0], preferred_element_type=jnp.float32)
    o_ref[0, 0] = res[:, :H]
    o_ref[1, 0] = res[:, H:]


def _hrk(h, W):
    return pl.pallas_call(
        _hrk_body,
        grid=(N // BN, R),
        in_specs=[
            pl.BlockSpec((BN, D), lambda nb, r: (nb, 0)),
            pl.BlockSpec((1, D, D), lambda nb, r: (r, 0, 0)),
        ],
        out_specs=pl.BlockSpec((NC, 1, BN, H), lambda nb, r: (0, r, nb, 0)),
        out_shape=jax.ShapeDtypeStruct((NC, R, N, H), jnp.float32),
    )(h, W)


def _mmk_body(h_ref, s_ref, b_ref, o_ref):
    o_ref[...] = (
        jnp.dot(h_ref[...], s_ref[...], preferred_element_type=jnp.float32)
        + b_ref[...]
    )


def _mmk(h, S, b2):
    return pl.pallas_call(
        _mmk_body,
        grid=(N // BN,),
        in_specs=[
            pl.BlockSpec((BN, D), lambda nb: (nb, 0)),
            pl.BlockSpec((D, D), lambda nb: (0, 0)),
            pl.BlockSpec((1, D), lambda nb: (0, 0)),
        ],
        out_specs=pl.BlockSpec((BN, D), lambda nb: (nb, 0)),
        out_shape=jax.ShapeDtypeStruct((N, D), jnp.float32),
    )(h, S, b2)


def _slk_body(agg_ref, slp_ref, s_ref, b_ref, h_ref, sl_ref):
    hcat = jnp.concatenate([agg_ref[0], agg_ref[1]], axis=1) + slp_ref[...]
    hb = jnp.maximum(hcat, 0.0)
    h_ref[...] = hb
    sl_ref[...] = (
        jnp.dot(hb, s_ref[...], preferred_element_type=jnp.float32) + b_ref[...]
    )


def _slk(agg, slp, S, b2):
    return pl.pallas_call(
        _slk_body,
        grid=(N // BN,),
        in_specs=[
            pl.BlockSpec((NC, BN, H), lambda nb: (0, nb, 0)),
            pl.BlockSpec((BN, D), lambda nb: (nb, 0)),
            pl.BlockSpec((D, D), lambda nb: (0, 0)),
            pl.BlockSpec((1, D), lambda nb: (0, 0)),
        ],
        out_specs=[
            pl.BlockSpec((BN, D), lambda nb: (nb, 0)),
            pl.BlockSpec((BN, D), lambda nb: (nb, 0)),
        ],
        out_shape=[
            jax.ShapeDtypeStruct((N, D), jnp.float32),
            jax.ShapeDtypeStruct((N, D), jnp.float32),
        ],
    )(agg, slp, S, b2)


def _fin_body(agg_ref, slp_ref, o_ref):
    nb = pl.program_id(0)
    hcat = jnp.concatenate([agg_ref[0], agg_ref[1]], axis=1) + slp_ref[...]
    hb = jnp.maximum(hcat, 0.0)
    part = jnp.sum(hb, axis=0, keepdims=True)

    @pl.when(nb == 0)
    def _():
        o_ref[...] = jnp.zeros_like(o_ref)

    o_ref[...] += part


def _fin(agg, slp):
    return pl.pallas_call(
        _fin_body,
        grid=(N // BN,),
        in_specs=[
            pl.BlockSpec((NC, BN, H), lambda nb: (0, nb, 0)),
            pl.BlockSpec((BN, D), lambda nb: (nb, 0)),
        ],
        out_specs=pl.BlockSpec((1, D), lambda nb: (0, 0)),
        out_shape=jax.ShapeDtypeStruct((1, D), jnp.float32),
    )(agg, slp)


def _sc_scatter(hr_flat, gidx_pad, dst_pad, zrows):
    mesh = plsc.VectorSubcoreMesh(core_axis_name="c", subcore_axis_name="s")

    @functools.partial(
        pl.kernel,
        mesh=mesh,
        out_type=jax.ShapeDtypeStruct((NC, ACC, H), jnp.float32),
        scratch_types=[
            pltpu.VMEM((CH,), jnp.int32),
            pltpu.VMEM((CH,), jnp.int32),
            pltpu.VMEM((CH, H), jnp.float32),
            pltpu.VMEM_SHARED((ACC, H), jnp.float32),
            pltpu.SemaphoreType.DMA,
        ],
    )
    def k(hr_hbm, gidx_hbm, dst_hbm, z_hbm, out_hbm, gix_v, dst_v, rows_v,
          acc_s, sem):
        cid = lax.axis_index("c")
        sid = lax.axis_index("s")
        row0 = sid * ROWS_PER_TILE
        pltpu.sync_copy(z_hbm.at[pl.ds(row0, ROWS_PER_TILE)],
                        acc_s.at[pl.ds(row0, ROWS_PER_TILE)])
        plsc.subcore_barrier()

        coff = cid * RN
        base0 = sid * P_SUB

        def body(j, carry):
            base = base0 + j * CH
            pltpu.sync_copy(gidx_hbm.at[pl.ds(base, CH)], gix_v)
            pltpu.sync_copy(dst_hbm.at[pl.ds(base, CH)], dst_v)
            for i in range(CH // 16):
                sl = pl.ds(i * 16, 16)
                gix_v[sl] = gix_v[sl] + coff
            pltpu.async_copy(hr_hbm.at[gix_v], rows_v, sem).wait()
            pltpu.sync_copy(rows_v, acc_s.at[dst_v], add=True)
            return carry

        lax.fori_loop(0, N_CH, body, 0)
        plsc.subcore_barrier()

        @pl.when(cid == 0)
        def _():
            pltpu.sync_copy(acc_s.at[pl.ds(row0, ROWS_PER_TILE)],
                            out_hbm.at[0, pl.ds(row0, ROWS_PER_TILE)])

        @pl.when(cid == 1)
        def _():
            pltpu.sync_copy(acc_s.at[pl.ds(row0, ROWS_PER_TILE)],
                            out_hbm.at[1, pl.ds(row0, ROWS_PER_TILE)])

    return k(hr_flat, gidx_pad, dst_pad, zrows)


def kernel(x, edge_index, edge_type, W0, S0, b0, W1, S1, b1, W2, S2, b2,
           W3, S3, b3):
    src, dst = edge_index[0], edge_index[1]
    gidx = edge_type * N + src
    pad = E_PAD - E
    gidx_p = jnp.concatenate([gidx, jnp.zeros((pad,), jnp.int32)])
    dst_p = jnp.concatenate([dst, jnp.full((pad,), N, jnp.int32)])
    zrows = jnp.zeros((ACC, H), jnp.float32)

    layers = ((W0, S0, b0), (W1, S1, b1), (W2, S2, b2), (W3, S3, b3))
    h = x
    sl = _mmk(x, S0, b0.reshape(1, D))
    agg = None
    for l in range(4):
        HR = _hrk(h, layers[l][0])
        agg = _sc_scatter(HR.reshape(NC * RN, H), gidx_p, dst_p, zrows)
        if l < 3:
            S_next, b_next = layers[l + 1][1], layers[l + 1][2]
            h, sl = _slk(agg, sl, S_next, b_next.reshape(1, D))
    out = _fin(agg, sl)
    return out.reshape(1, 1, D)
